# Initial kernel scaffold; baseline (speedup 1.0000x reference)
import jax
import jax.numpy as jnp
from jax.experimental import pallas as pl


def kernel(X, edge_index, edge_attr, W_node, W_edge, W_self, b_self, gamma, beta):
    raise NotImplementedError("write your pallas kernel here")


# speedup vs baseline: 35.0400x; 35.0400x over previous
import functools

import jax
import jax.numpy as jnp
from jax import lax
from jax.experimental import pallas as pl
from jax.experimental.pallas import tpu as pltpu
from jax.experimental.pallas import tpu_sc as plsc

B, V, F_DIM, E, O = 2, 10000, 128, 160000, 128
NS = 16
EPT = E // NS
K = 80
NCHUNK = EPT // K
VPT = V // NS
VPAD = 640
SPAN = 640
TAIL = V - (NS - 1) * SPAN
EPS = 1e-5


def _sc_aggregate(xflat, srcg, dst4, attr5):
    mesh = plsc.VectorSubcoreMesh(
        core_axis_name="c", subcore_axis_name="s", num_cores=2,
        num_subcores=NS)

    @functools.partial(
        pl.kernel,
        out_type=[
            jax.ShapeDtypeStruct((B * V, F_DIM), jnp.float32),
            jax.ShapeDtypeStruct((B, NS, 1, VPAD), jnp.float32),
        ],
        mesh=mesh,
        scratch_types=[
            pltpu.VMEM((K,), jnp.int32),
            pltpu.VMEM((K,), jnp.int32),
            pltpu.VMEM((K,), jnp.float32),
            pltpu.VMEM((K, F_DIM), jnp.float32),
            pltpu.VMEM((NS * VPAD,), jnp.float32),
            pltpu.VMEM((VPAD,), jnp.float32),
            pltpu.VMEM((VPAD,), jnp.float32),
            pltpu.VMEM_SHARED((V, F_DIM), jnp.float32),
            pltpu.VMEM_SHARED((NS, 1, NS * VPAD), jnp.float32),
            pltpu.SemaphoreType.DMA,
        ],
        compiler_params=pltpu.CompilerParams(needs_layout_passes=False),
    )
    def agg(xflat_hbm, srcg_hbm, dst4_hbm, attr5_hbm, a_out, s_out,
            src_c, dst_c, attr_c, rows_v, sp, sacc, stmp, sh_a, sh_s,
            sem):
        c = lax.axis_index("c")
        s = lax.axis_index("s")
        zero16 = jnp.zeros((16,), jnp.float32)

        def zero_rows(i, _):
            rows_v[i // 8, pl.ds((i % 8) * 16, 16)] = zero16
            return 0
        lax.fori_loop(0, K * F_DIM // 16, zero_rows, 0)

        def zero_sp(i, _):
            sp[pl.ds(i * 16, 16)] = zero16
            return 0
        lax.fori_loop(0, NS * VPAD // 16, zero_sp, 0)

        nq = jnp.where(s == NS - 1, 5, 8)
        base = s * SPAN

        def zero_sh(q, _):
            pltpu.sync_copy(rows_v, sh_a.at[pl.ds(base + q * K, K)])
            return 0
        lax.fori_loop(0, nq, zero_sh, 0)
        plsc.subcore_barrier()

        def edge_body(j, _):
            pltpu.sync_copy(srcg_hbm.at[c, s, j, 0], src_c)
            pltpu.sync_copy(dst4_hbm.at[s, j, 0], dst_c)
            pltpu.sync_copy(attr5_hbm.at[c, s, j, 0], attr_c)
            pltpu.async_copy(xflat_hbm.at[src_c], rows_v, sem).wait()
            pltpu.sync_copy(rows_v, sh_a.at[dst_c], add=True)
            for u in range(K // 16):
                d = dst_c[pl.ds(u * 16, 16)]
                a = attr_c[pl.ds(u * 16, 16)]
                idx = d + 15 * (d // VPT)
                plsc.addupdate_scatter(sp, [idx], a)
            return 0
        lax.fori_loop(0, NCHUNK, edge_body, 0)

        pltpu.sync_copy(sp, sh_s.at[s, 0])
        plsc.subcore_barrier()

        pltpu.sync_copy(sh_s.at[0, 0, pl.ds(s * VPAD, VPAD)], sacc)

        def red_body(u, _):
            pltpu.sync_copy(sh_s.at[u, 0, pl.ds(s * VPAD, VPAD)], stmp)

            def add_body(k2, _):
                sacc[pl.ds(k2 * 16, 16)] = (
                    sacc[pl.ds(k2 * 16, 16)] + stmp[pl.ds(k2 * 16, 16)])
                return 0
            lax.fori_loop(0, VPAD // 16, add_body, 0)
            return 0
        lax.fori_loop(1, NS, red_body, 0)
        pltpu.sync_copy(sacc, s_out.at[c, s, 0])

        @pl.when(s < NS - 1)
        def _():
            pltpu.sync_copy(
                sh_a.at[pl.ds(s * SPAN, SPAN)],
                a_out.at[pl.ds(c * V + s * SPAN, SPAN)])

        @pl.when(s == NS - 1)
        def _():
            pltpu.sync_copy(
                sh_a.at[pl.ds((NS - 1) * SPAN, TAIL)],
                a_out.at[pl.ds(c * V + (NS - 1) * SPAN, TAIL)])

    return agg(xflat, srcg, dst4, attr5)


VB = 1000
NBLK = B * V // VB


def _tc_dense(xflat, aflat, sflat, w_self, w_node, we_row, b_row):
    def body(x_ref, a_ref, s_ref, ws_ref, wn_ref, we_ref, b_ref,
             h_ref, sum_ref, sq_ref):
        nt = (((1,), (1,)), ((), ()))
        h = lax.dot_general(x_ref[...], ws_ref[...], nt,
                            preferred_element_type=jnp.float32)
        h = h + lax.dot_general(a_ref[...], wn_ref[...], nt,
                                preferred_element_type=jnp.float32)
        h = h + s_ref[...] * we_ref[...]
        h = h + b_ref[...]
        h_ref[...] = h

        @pl.when(pl.program_id(0) == 0)
        def _():
            sum_ref[...] = jnp.zeros_like(sum_ref)
            sq_ref[...] = jnp.zeros_like(sq_ref)
        sum_ref[...] += jnp.sum(h, axis=0, keepdims=True)
        sq_ref[...] += jnp.sum(h * h, axis=0, keepdims=True)

    full = lambda shape: pl.BlockSpec(shape, lambda i: (0, 0))
    return pl.pallas_call(
        body,
        grid=(NBLK,),
        in_specs=[
            pl.BlockSpec((VB, F_DIM), lambda i: (i, 0)),
            pl.BlockSpec((VB, F_DIM), lambda i: (i, 0)),
            pl.BlockSpec((VB, 1), lambda i: (i, 0)),
            full((O, F_DIM)),
            full((O, F_DIM)),
            full((1, O)),
            full((1, O)),
        ],
        out_specs=[
            pl.BlockSpec((VB, O), lambda i: (i, 0)),
            full((1, O)),
            full((1, O)),
        ],
        out_shape=[
            jax.ShapeDtypeStruct((B * V, O), jnp.float32),
            jax.ShapeDtypeStruct((1, O), jnp.float32),
            jax.ShapeDtypeStruct((1, O), jnp.float32),
        ],
    )(xflat, aflat, sflat, w_self, w_node, we_row, b_row)


def _tc_norm(h, hsum, hsq, g_row, beta_row):
    def body(h_ref, sum_ref, sq_ref, g_ref, be_ref, o_ref):
        n = float(B * V)
        mean = sum_ref[...] / n
        var = sq_ref[...] / n - mean * mean
        scale = g_ref[...] * lax.rsqrt(var + EPS)
        shift = be_ref[...] - mean * scale
        o_ref[...] = jnp.maximum(h_ref[...] * scale + shift, 0.0)

    full = lambda shape: pl.BlockSpec(shape, lambda i: (0, 0))
    return pl.pallas_call(
        body,
        grid=(NBLK,),
        in_specs=[
            pl.BlockSpec((VB, O), lambda i: (i, 0)),
            full((1, O)),
            full((1, O)),
            full((1, O)),
            full((1, O)),
        ],
        out_specs=pl.BlockSpec((VB, O), lambda i: (i, 0)),
        out_shape=jax.ShapeDtypeStruct((B * V, O), jnp.float32),
    )(h, hsum, hsq, g_row, beta_row)


@jax.jit
def kernel(X, edge_index, edge_attr, W_node, W_edge, W_self, b_self, gamma,
           beta):
    ei = edge_index.astype(jnp.int32)
    src = ei[:, 0]
    dst = ei[:, 1]
    srcg = (src[None, :] + (jnp.arange(B, dtype=jnp.int32) * V)[:, None])
    srcg = srcg.reshape(B, NS, NCHUNK, 1, K)
    dst4 = dst.reshape(NS, NCHUNK, 1, K)
    attr5 = edge_attr.reshape(B, NS, NCHUNK, 1, K)
    xflat = X.reshape(B * V, F_DIM)

    aflat, s_pad = _sc_aggregate(xflat, srcg, dst4, attr5)
    sflat = s_pad[:, :, 0, :VPT].reshape(B * V, 1)

    h, hsum, hsq = _tc_dense(
        xflat, aflat, sflat, W_self, W_node,
        W_edge.reshape(1, O), b_self.reshape(1, O))
    out = _tc_norm(h, hsum, hsq, gamma.reshape(1, O), beta.reshape(1, O))
    return out.reshape(B, V, O)


# speedup vs baseline: 62.1591x; 1.7739x over previous
import functools

import jax
import jax.numpy as jnp
from jax import lax
from jax.experimental import pallas as pl
from jax.experimental.pallas import tpu as pltpu
from jax.experimental.pallas import tpu_sc as plsc

B, V, F_DIM, E, O = 2, 10000, 128, 160000, 128
NS = 16
EPT = E // NS
K = 80
NCHUNK = EPT // K
VPT = V // NS
VPAD = 640
SPAN = 640
TAIL = V - (NS - 1) * SPAN
EPS = 1e-5


def _sc_aggregate(xflat, pack):
    mesh = plsc.VectorSubcoreMesh(
        core_axis_name="c", subcore_axis_name="s", num_cores=2,
        num_subcores=NS)

    @functools.partial(
        pl.kernel,
        out_type=[
            jax.ShapeDtypeStruct((B * V, F_DIM), jnp.float32),
            jax.ShapeDtypeStruct((B, NS, 1, VPAD), jnp.float32),
        ],
        mesh=mesh,
        scratch_types=[
            pltpu.VMEM((3, K), jnp.int32),
            pltpu.VMEM((3, K), jnp.int32),
            pltpu.VMEM((K,), jnp.int32),
            pltpu.VMEM((K,), jnp.int32),
            pltpu.VMEM((K, F_DIM), jnp.float32),
            pltpu.VMEM((K, F_DIM), jnp.float32),
            pltpu.VMEM((NS * VPAD,), jnp.float32),
            pltpu.VMEM((VPAD,), jnp.float32),
            pltpu.VMEM((VPAD,), jnp.float32),
            pltpu.VMEM_SHARED((V, F_DIM), jnp.float32),
            pltpu.VMEM_SHARED((NS, 1, NS * VPAD), jnp.float32),
            pltpu.SemaphoreType.DMA,
            pltpu.SemaphoreType.DMA,
            pltpu.SemaphoreType.DMA,
            pltpu.SemaphoreType.DMA,
            pltpu.SemaphoreType.DMA,
            pltpu.SemaphoreType.DMA,
        ],
        compiler_params=pltpu.CompilerParams(needs_layout_passes=False),
    )
    def agg(xflat_hbm, pack_hbm, a_out, s_out,
            pk0, pk1, di0, di1, rw0, rw1, sp, sacc, stmp, sh_a, sh_s,
            is0, is1, gs0, gs1, ss0, ss1):
        c = lax.axis_index("c")
        s = lax.axis_index("s")
        zero16 = jnp.zeros((16,), jnp.float32)
        pk = (pk0, pk1)
        di = (di0, di1)
        rw = (rw0, rw1)
        isem = (is0, is1)
        gsem = (gs0, gs1)
        ssem = (ss0, ss1)

        def idx_load(j, slot):
            pltpu.async_copy(pack_hbm.at[c, s, j], pk[slot], isem[slot])

        def idx_wait(slot):
            pltpu.make_async_copy(pack_hbm.at[c, s, 0], pk[slot],
                                  isem[slot]).wait()

        def gather_start(slot):
            pltpu.async_copy(xflat_hbm.at[pk[slot].at[0]], rw[slot],
                             gsem[slot])

        def gather_wait(slot):
            pltpu.make_async_copy(xflat_hbm.at[pk[slot].at[0]], rw[slot],
                                  gsem[slot]).wait()

        def scat_start(slot):
            pltpu.async_copy(rw[slot], sh_a.at[di[slot]], ssem[slot],
                             add=True)

        def scat_wait(slot):
            pltpu.make_async_copy(rw[slot], sh_a.at[di[slot]],
                                  ssem[slot]).wait()

        def sp_accum(slot):
            for u in range(K // 16):
                d = pk[slot][1, pl.ds(u * 16, 16)]
                abits = pk[slot][2, pl.ds(u * 16, 16)]
                di[slot][pl.ds(u * 16, 16)] = d
                idx = d + 15 * (d // VPT)
                plsc.addupdate_scatter(sp, [idx],
                                       plsc.bitcast(abits, jnp.float32))

        idx_load(0, 0)
        idx_load(1, 1)

        def zero_rows(i, _):
            rw0[i // 8, pl.ds((i % 8) * 16, 16)] = zero16
            return 0
        lax.fori_loop(0, K * F_DIM // 16, zero_rows, 0)

        def zero_sp(i, _):
            sp[pl.ds(i * 16, 16)] = zero16
            return 0
        lax.fori_loop(0, NS * VPAD // 16, zero_sp, 0)

        nq = jnp.where(s == NS - 1, 5, 8)
        base = s * SPAN

        def zero_sh(q, _):
            pltpu.sync_copy(rw0, sh_a.at[pl.ds(base + q * K, K)])
            return 0
        lax.fori_loop(0, nq, zero_sh, 0)
        plsc.subcore_barrier()

        idx_wait(0)
        gather_start(0)

        def chunk_step(j, slot):
            other = 1 - slot
            gather_wait(slot)
            sp_accum(slot)
            scat_start(slot)

            @pl.when(j < NCHUNK - 2)
            def _():
                idx_load(j + 2, slot)

            @pl.when(j > 0)
            def _():
                scat_wait(other)

            @pl.when(j < NCHUNK - 1)
            def _():
                idx_wait(other)
                gather_start(other)

        def pair_body(p, _):
            chunk_step(2 * p, 0)
            chunk_step(2 * p + 1, 1)
            return 0
        lax.fori_loop(0, (NCHUNK - 1) // 2, pair_body, 0)
        chunk_step(NCHUNK - 1, 0)
        scat_wait(0)

        pltpu.sync_copy(sp, sh_s.at[s, 0])
        plsc.subcore_barrier()

        pltpu.sync_copy(sh_s.at[0, 0, pl.ds(s * VPAD, VPAD)], sacc)

        def red_body(u, _):
            pltpu.sync_copy(sh_s.at[u, 0, pl.ds(s * VPAD, VPAD)], stmp)

            def add_body(k2, _):
                sacc[pl.ds(k2 * 16, 16)] = (
                    sacc[pl.ds(k2 * 16, 16)] + stmp[pl.ds(k2 * 16, 16)])
                return 0
            lax.fori_loop(0, VPAD // 16, add_body, 0)
            return 0
        lax.fori_loop(1, NS, red_body, 0)
        pltpu.sync_copy(sacc, s_out.at[c, s, 0])

        @pl.when(s < NS - 1)
        def _():
            pltpu.sync_copy(
                sh_a.at[pl.ds(s * SPAN, SPAN)],
                a_out.at[pl.ds(c * V + s * SPAN, SPAN)])

        @pl.when(s == NS - 1)
        def _():
            pltpu.sync_copy(
                sh_a.at[pl.ds((NS - 1) * SPAN, TAIL)],
                a_out.at[pl.ds(c * V + (NS - 1) * SPAN, TAIL)])

    return agg(xflat, pack)


VB = 1000
NBLK = B * V // VB


def _tc_dense(xflat, aflat, sflat, w_self, w_node, we_row, b_row):
    def body(x_ref, a_ref, s_ref, ws_ref, wn_ref, we_ref, b_ref,
             h_ref, sum_ref, sq_ref):
        nt = (((1,), (1,)), ((), ()))
        h = lax.dot_general(x_ref[...], ws_ref[...], nt,
                            preferred_element_type=jnp.float32)
        h = h + lax.dot_general(a_ref[...], wn_ref[...], nt,
                                preferred_element_type=jnp.float32)
        h = h + s_ref[...] * we_ref[...]
        h = h + b_ref[...]
        h_ref[...] = h

        @pl.when(pl.program_id(0) == 0)
        def _():
            sum_ref[...] = jnp.zeros_like(sum_ref)
            sq_ref[...] = jnp.zeros_like(sq_ref)
        sum_ref[...] += jnp.sum(h, axis=0, keepdims=True)
        sq_ref[...] += jnp.sum(h * h, axis=0, keepdims=True)

    full = lambda shape: pl.BlockSpec(shape, lambda i: (0, 0))
    return pl.pallas_call(
        body,
        grid=(NBLK,),
        in_specs=[
            pl.BlockSpec((VB, F_DIM), lambda i: (i, 0)),
            pl.BlockSpec((VB, F_DIM), lambda i: (i, 0)),
            pl.BlockSpec((VB, 1), lambda i: (i, 0)),
            full((O, F_DIM)),
            full((O, F_DIM)),
            full((1, O)),
            full((1, O)),
        ],
        out_specs=[
            pl.BlockSpec((VB, O), lambda i: (i, 0)),
            full((1, O)),
            full((1, O)),
        ],
        out_shape=[
            jax.ShapeDtypeStruct((B * V, O), jnp.float32),
            jax.ShapeDtypeStruct((1, O), jnp.float32),
            jax.ShapeDtypeStruct((1, O), jnp.float32),
        ],
    )(xflat, aflat, sflat, w_self, w_node, we_row, b_row)


def _tc_norm(h, hsum, hsq, g_row, beta_row):
    def body(h_ref, sum_ref, sq_ref, g_ref, be_ref, o_ref):
        n = float(B * V)
        mean = sum_ref[...] / n
        var = sq_ref[...] / n - mean * mean
        scale = g_ref[...] * lax.rsqrt(var + EPS)
        shift = be_ref[...] - mean * scale
        o_ref[...] = jnp.maximum(h_ref[...] * scale + shift, 0.0)

    full = lambda shape: pl.BlockSpec(shape, lambda i: (0, 0))
    return pl.pallas_call(
        body,
        grid=(NBLK,),
        in_specs=[
            pl.BlockSpec((VB, O), lambda i: (i, 0)),
            full((1, O)),
            full((1, O)),
            full((1, O)),
            full((1, O)),
        ],
        out_specs=pl.BlockSpec((VB, O), lambda i: (i, 0)),
        out_shape=jax.ShapeDtypeStruct((B * V, O), jnp.float32),
    )(h, hsum, hsq, g_row, beta_row)


@jax.jit
def kernel(X, edge_index, edge_attr, W_node, W_edge, W_self, b_self, gamma,
           beta):
    ei = edge_index.astype(jnp.int32)
    src = ei[:, 0]
    dst = ei[:, 1]
    srcg = src[None, :] + (jnp.arange(B, dtype=jnp.int32) * V)[:, None]
    dstb = jnp.broadcast_to(dst[None, :], (B, E))
    abits = jax.lax.bitcast_convert_type(edge_attr, jnp.int32)
    pack = jnp.stack([srcg.reshape(B, NS, NCHUNK, K),
                      dstb.reshape(B, NS, NCHUNK, K),
                      abits.reshape(B, NS, NCHUNK, K)], axis=3)
    xflat = X.reshape(B * V, F_DIM)

    aflat, s_pad = _sc_aggregate(xflat, pack)
    sflat = s_pad[:, :, 0, :VPT].reshape(B * V, 1)

    h, hsum, hsq = _tc_dense(
        xflat, aflat, sflat, W_self, W_node,
        W_edge.reshape(1, O), b_self.reshape(1, O))
    out = _tc_norm(h, hsum, hsq, gamma.reshape(1, O), beta.reshape(1, O))
    return out.reshape(B, V, O)
